# baseline (speedup 1.0000x reference)
import functools

import jax
import jax.numpy as jnp
from jax import lax
from jax.experimental import pallas as pl
from jax.experimental.pallas import tpu as pltpu
from jax.experimental.pallas import tpu_sc as plsc

_VOCAB = 1000000
_EMBED = 64
_B = 4096
_L = 200
_N = _B * _L

_NC = 2
_NS = 16
_NW = _NC * _NS
_PER_W = _N // _NW
_CHUNK = 512
_STEPS = _PER_W // _CHUNK

_mesh = plsc.VectorSubcoreMesh(core_axis_name="c", subcore_axis_name="s")


@functools.partial(
    pl.kernel,
    mesh=_mesh,
    out_type=jax.ShapeDtypeStruct((_N, _EMBED), jnp.float32),
    scratch_types=[
        pltpu.VMEM((_CHUNK,), jnp.int32),
        pltpu.VMEM((_CHUNK, _EMBED), jnp.float32),
        pltpu.SemaphoreType.DMA,
    ],
    compiler_params=pltpu.CompilerParams(use_tc_tiling_on_sc=False),
)
def _emb_lookup(x_hbm, table_hbm, out_hbm, idx_v, rows_v, sem):
    wid = lax.axis_index("s") * _NC + lax.axis_index("c")
    base = wid * _PER_W

    def step(ci, carry):
        off = base + ci * _CHUNK
        pltpu.sync_copy(x_hbm.at[pl.ds(off, _CHUNK)], idx_v)
        pltpu.async_copy(table_hbm.at[idx_v], rows_v, sem).wait()

        def scale_row(i, c):
            for j in range(_EMBED // 16):
                sl = rows_v[i, pl.ds(j * 16, 16)]
                rows_v[i, pl.ds(j * 16, 16)] = sl * 0.125
            return c

        lax.fori_loop(0, _CHUNK, scale_row, 0)
        pltpu.sync_copy(rows_v, out_hbm.at[pl.ds(off, _CHUNK)])
        return carry

    lax.fori_loop(0, _STEPS, step, 0)


def kernel(x, table):
    xf = x.reshape(_N).astype(jnp.int32)
    out = _emb_lookup(xf, table)
    return out.reshape(_B, _L, _EMBED)


# speedup vs baseline: 1.1315x; 1.1315x over previous
import functools

import jax
import jax.numpy as jnp
from jax import lax
from jax.experimental import pallas as pl
from jax.experimental.pallas import tpu as pltpu
from jax.experimental.pallas import tpu_sc as plsc

_VOCAB = 1000000
_EMBED = 64
_B = 4096
_L = 200

_NC = 2
_NS = 16
_NW = _NC * _NS
_ROWS_W = _B // _NW
_R = 4
_STEPS = _ROWS_W // _R

_mesh = plsc.VectorSubcoreMesh(core_axis_name="c", subcore_axis_name="s")


@functools.partial(
    pl.kernel,
    mesh=_mesh,
    out_type=jax.ShapeDtypeStruct((_B, _L, _EMBED), jnp.float32),
    scratch_types=[
        pltpu.VMEM((2, _R, _L), jnp.int32),
        pltpu.VMEM((2, _R, _L, _EMBED), jnp.float32),
        pltpu.SemaphoreType.DMA,
        pltpu.SemaphoreType.DMA,
        pltpu.SemaphoreType.DMA,
        pltpu.SemaphoreType.DMA,
    ],
    compiler_params=pltpu.CompilerParams(use_tc_tiling_on_sc=False),
)
def _emb_lookup(x_hbm, table_hbm, out_hbm, idx2, rows2, g0, g1, s0, s1):
    wid = lax.axis_index("s") * _NC + lax.axis_index("c")
    base = wid * _ROWS_W
    gsems = (g0, g1)
    ssems = (s0, s1)

    def start_chunk(ci, b):
        r0 = base + ci * _R
        pltpu.sync_copy(x_hbm.at[pl.ds(r0, _R)], idx2.at[b])
        for k in range(_R):
            pltpu.make_async_copy(
                table_hbm.at[idx2.at[b, k]], rows2.at[b, k], gsems[b]
            ).start()

    def wait_chunk(ci, b):
        r0 = base + ci * _R
        for k in range(_R):
            pltpu.make_async_copy(
                table_hbm.at[idx2.at[b, k]], rows2.at[b, k], gsems[b]
            ).wait()

    def start_store(ci, b):
        r0 = base + ci * _R
        pltpu.make_async_copy(rows2.at[b], out_hbm.at[pl.ds(r0, _R)], ssems[b]).start()

    def wait_store(ci, b):
        r0 = base + ci * _R
        pltpu.make_async_copy(rows2.at[b], out_hbm.at[pl.ds(r0, _R)], ssems[b]).wait()

    def scale(b):
        def body(i, c):
            for k in range(_R):
                for j in range(_EMBED // 16):
                    sl = rows2[b, k, i, pl.ds(j * 16, 16)]
                    rows2[b, k, i, pl.ds(j * 16, 16)] = sl * 0.125
            return c

        lax.fori_loop(0, _L, body, 0)

    start_chunk(0, 0)

    def step(it, carry):
        for b in range(2):
            ci = it * 2 + b
            nb = b ^ 1
            nci = ci + 1

            @pl.when(nci < _STEPS)
            def _prefetch():
                @pl.when(nci >= 2)
                def _drain():
                    wait_store(nci - 2, nb)

                start_chunk(nci, nb)

            wait_chunk(ci, b)
            scale(b)
            start_store(ci, b)
        return carry

    lax.fori_loop(0, _STEPS // 2, step, 0)

    wait_store(_STEPS - 2, 0)
    wait_store(_STEPS - 1, 1)


def kernel(x, table):
    return _emb_lookup(x, table)
